# Initial kernel scaffold; baseline (speedup 1.0000x reference)
import functools

import jax
import jax.numpy as jnp
from jax import lax
from jax.experimental import pallas as pl
from jax.experimental.pallas import tpu as pltpu
from jax.experimental.pallas import tpu_sc as plsc

D = 64
NC, NS = 2, 16
NW = NC * NS
IBLK = 128
K = 4
CH = K * IBLK


def _make_gather(n_rows: int):
    r_per_w = n_rows // NW
    n_chunks = r_per_w // CH
    mesh = plsc.VectorSubcoreMesh(core_axis_name="c", subcore_axis_name="s")

    @functools.partial(
        pl.kernel,
        mesh=mesh,
        out_type=jax.ShapeDtypeStruct((n_rows, D), jnp.float32),
        scratch_types=[
            pltpu.VMEM((K, IBLK), jnp.int32),
            pltpu.VMEM((CH, D), jnp.float32),
            pltpu.SemaphoreType.DMA,
        ],
    )
    def gather(idx_hbm, table_hbm, out_hbm, idx_v, rows_v, sem):
        wid = lax.axis_index("s") * NC + lax.axis_index("c")
        row_base = wid * r_per_w
        iblk_base = row_base // IBLK

        def chunk_body(c, carry):
            off = row_base + c * CH
            pltpu.sync_copy(idx_hbm.at[pl.ds(iblk_base + c * K, K)], idx_v)
            copies = [
                pltpu.async_copy(
                    table_hbm.at[idx_v.at[j]],
                    rows_v.at[pl.ds(j * IBLK, IBLK)],
                    sem,
                )
                for j in range(K)
            ]
            for cp in copies:
                cp.wait()
            pltpu.sync_copy(rows_v, out_hbm.at[pl.ds(off, CH)])
            return carry

        lax.fori_loop(0, n_chunks, chunk_body, 0)

    return gather


def kernel(input_ids, table):
    b, h = input_ids.shape
    n = b * h
    idx2d = input_ids.reshape(n // IBLK, IBLK).astype(jnp.int32)
    out = _make_gather(n)(idx2d, table)
    return out.reshape(b, h, D)


# speedup vs baseline: 1.8733x; 1.8733x over previous
import functools

import jax
import jax.numpy as jnp
from jax import lax
from jax.experimental import pallas as pl
from jax.experimental.pallas import tpu as pltpu
from jax.experimental.pallas import tpu_sc as plsc

D = 64
NC, NS = 2, 16
NW = NC * NS
IBLK = 128
K = 4
CH = K * IBLK
NBUF = 2


def _make_gather(n_rows: int):
    r_per_w = n_rows // NW
    n_chunks = r_per_w // CH
    n_outer = n_chunks // NBUF
    mesh = plsc.VectorSubcoreMesh(core_axis_name="c", subcore_axis_name="s")

    @functools.partial(
        pl.kernel,
        mesh=mesh,
        out_type=jax.ShapeDtypeStruct((n_rows, D), jnp.float32),
        scratch_types=[
            pltpu.VMEM((NBUF, K, IBLK), jnp.int32),
            pltpu.VMEM((NBUF, CH, D), jnp.float32),
            [pltpu.SemaphoreType.DMA] * NBUF,
            [pltpu.SemaphoreType.DMA] * NBUF,
        ],
        compiler_params=pltpu.CompilerParams(use_tc_tiling_on_sc=False),
    )
    def gather(idx_hbm, table_hbm, out_hbm, idx_v, rows_v, gsems, wsems):
        wid = lax.axis_index("s") * NC + lax.axis_index("c")
        row_base = wid * r_per_w
        iblk_base = row_base // IBLK

        def start_gather(c, b):
            ioff = pl.multiple_of(iblk_base + c * K, K)
            pltpu.sync_copy(idx_hbm.at[pl.ds(ioff, K)], idx_v.at[b])
            for j in range(K):
                pltpu.async_copy(
                    table_hbm.at[idx_v.at[b, j]],
                    rows_v.at[b, pl.ds(j * IBLK, IBLK)],
                    gsems[b],
                )

        def wait_gather(b):
            pltpu.make_async_copy(
                table_hbm.at[pl.ds(0, CH)], rows_v.at[b], gsems[b]
            ).wait()

        def start_write(c, b):
            off = pl.multiple_of(row_base + c * CH, CH)
            pltpu.async_copy(rows_v.at[b], out_hbm.at[pl.ds(off, CH)], wsems[b])

        def wait_write(b):
            pltpu.make_async_copy(
                rows_v.at[b], out_hbm.at[pl.ds(0, CH)], wsems[b]
            ).wait()

        for b in range(NBUF):
            start_gather(b, b)

        def outer_body(i, carry):
            c0 = i * NBUF
            for b in range(NBUF):
                wait_gather(b)
                start_write(c0 + b, b)
            for b in range(NBUF):
                wait_write(b)
                start_gather(c0 + b + NBUF, b)
            return carry

        lax.fori_loop(0, n_outer - 1, outer_body, 0)

        c0 = (n_outer - 1) * NBUF
        for b in range(NBUF):
            wait_gather(b)
            start_write(c0 + b, b)
        for b in range(NBUF):
            wait_write(b)

    return gather


def kernel(input_ids, table):
    b, h = input_ids.shape
    n = b * h
    idx2d = input_ids.reshape(n // IBLK, IBLK).astype(jnp.int32)
    out = _make_gather(n)(idx2d, table)
    return out.reshape(b, h, D)
